# Initial kernel scaffold; baseline (speedup 1.0000x reference)
import jax
import jax.numpy as jnp
from jax.experimental import pallas as pl


def kernel(x, emb, W, b, gamma, beta):
    raise NotImplementedError("write your pallas kernel here")


# speedup vs baseline: 16.9991x; 16.9991x over previous
import functools

import jax
import jax.numpy as jnp
from jax import lax
from jax.experimental import pallas as pl
from jax.experimental.pallas import tpu as pltpu
from jax.experimental.pallas import tpu_sc as plsc

EMB_SIZE = 1000000
EMB_UNITS = 16
OUT_UNITS = 128
B = 16384
F = 26

NC, NS = 2, 16
NW = NC * NS
N_IDX = B * F
PER_W = N_IDX // NW
CHUNK = 1024
NCHUNK = PER_W // CHUNK

_mesh = plsc.VectorSubcoreMesh(
    core_axis_name="c", subcore_axis_name="s", num_cores=NC, num_subcores=NS
)


@functools.partial(
    pl.kernel,
    out_type=jax.ShapeDtypeStruct((N_IDX, EMB_UNITS), jnp.float32),
    mesh=_mesh,
    scratch_types=[
        pltpu.VMEM((PER_W,), jnp.int32),
        pltpu.VMEM((CHUNK, EMB_UNITS), jnp.float32),
        pltpu.VMEM((CHUNK, EMB_UNITS), jnp.float32),
        pltpu.SemaphoreType.DMA,
        pltpu.SemaphoreType.DMA,
    ],
    compiler_params=pltpu.CompilerParams(use_tc_tiling_on_sc=False),
)
def _sc_gather(idx_hbm, emb_hbm, out_hbm, idx_v, buf0, buf1, sem0, sem1):
    wid = lax.axis_index("s") * NC + lax.axis_index("c")
    base = wid * PER_W
    pltpu.sync_copy(idx_hbm.at[pl.ds(base, PER_W)], idx_v)
    bufs = (buf0, buf1)
    sems = (sem0, sem1)
    cps = [None, None]
    for k in range(NCHUNK):
        j = k % 2
        cps[j] = pltpu.async_copy(
            emb_hbm.at[idx_v.at[pl.ds(k * CHUNK, CHUNK)]], bufs[j], sems[j]
        )
        if k > 0:
            cps[1 - j].wait()
            pltpu.sync_copy(
                bufs[1 - j], out_hbm.at[pl.ds(base + (k - 1) * CHUNK, CHUNK)]
            )
    j = (NCHUNK - 1) % 2
    cps[j].wait()
    pltpu.sync_copy(bufs[j], out_hbm.at[pl.ds(base + (NCHUNK - 1) * CHUNK, CHUNK)])


BLK = 2048


def _tc_body(e_ref, w_ref, p_ref, o_ref):
    h = jnp.dot(e_ref[...], w_ref[...], preferred_element_type=jnp.float32)
    h = h + p_ref[0, :]
    mu = jnp.mean(h, axis=-1, keepdims=True)
    var = jnp.mean((h - mu) * (h - mu), axis=-1, keepdims=True)
    hn = (h - mu) * lax.rsqrt(var + 1e-5)
    y = hn * p_ref[1, :] + p_ref[2, :]
    o_ref[...] = y * jax.nn.sigmoid(y)


def kernel(x, emb, W, b, gamma, beta):
    idx = x.astype(jnp.int32).reshape(-1)
    e2 = _sc_gather(idx, emb)
    e = e2.reshape(B, F * EMB_UNITS)
    params = jnp.stack([b, gamma, beta])
    y = pl.pallas_call(
        _tc_body,
        grid=(B // BLK,),
        in_specs=[
            pl.BlockSpec((BLK, F * EMB_UNITS), lambda i: (i, 0)),
            pl.BlockSpec((F * EMB_UNITS, OUT_UNITS), lambda i: (0, 0)),
            pl.BlockSpec((3, OUT_UNITS), lambda i: (0, 0)),
        ],
        out_specs=pl.BlockSpec((BLK, OUT_UNITS), lambda i: (i, 0)),
        out_shape=jax.ShapeDtypeStruct((B, OUT_UNITS), jnp.float32),
    )(e, W, params)
    return y
